# baseline (speedup 1.0000x reference)
import functools

import jax
import jax.numpy as jnp
from jax import lax
from jax.experimental import pallas as pl
from jax.experimental.pallas import tpu as pltpu
from jax.experimental.pallas import tpu_sc as plsc

_NS = 64
_NI = 128
_NT = _NS + _NI
_NP = 256
_B = 256

_F32 = jnp.float32
_I32 = jnp.int32

_N_RAYS = 16384
_NW = 32
_RPW = _N_RAYS // _NW
_CH = 256
_L = 16


def _sc_resample_body(rpw, z_hbm, w_hbm, u_hbm, zs_hbm,
                      zbuf, wbuf, ubuf, tbuf, mbuf, zsbuf, sem):
    nc = 2
    wid = lax.axis_index("s") * nc + lax.axis_index("c")
    iv = lax.iota(_I32, _L)
    inf = jnp.full((_L,), jnp.inf, _F32)

    pltpu.sync_copy(u_hbm, ubuf)
    u_vecs = [ubuf[pl.ds(c * _L, _L)] for c in range(_NI // _L)]

    def do_chunk(chunk, _):
        base = wid * rpw + chunk * _CH
        pltpu.sync_copy(z_hbm.at[pl.ds(base * _NS, _CH * _NS)],
                        zbuf.at[pl.ds(0, _CH * _NS)])
        pltpu.sync_copy(w_hbm.at[pl.ds(base * _NS, _CH * _NS)],
                        wbuf.at[pl.ds(0, _CH * _NS)])

        def do_ray(r, _):
            rb = r * _NS
            wv = []
            for t in range(4):
                wt = plsc.load_gather(wbuf, [jnp.broadcast_to(rb + 1 + t * _L, (_L,)) + iv])
                wt = wt + 1e-5
                if t == 3:
                    wt = jnp.where(iv < 14, wt, 0.0)
                wv.append(wt)
            s0 = jnp.sum(wv[0])
            s1 = jnp.sum(wv[1])
            s2 = jnp.sum(wv[2])
            s3 = jnp.sum(wv[3])
            c0 = plsc.cumsum(wv[0])
            c1 = plsc.cumsum(wv[1]) + s0
            c2 = plsc.cumsum(wv[2]) + (s0 + s1)
            c3 = plsc.cumsum(wv[3]) + (s0 + s1 + s2)
            s_tot = s0 + s1 + s2 + s3
            plsc.store_scatter(tbuf, [iv], jnp.where(iv == 0, 0.0, inf))
            plsc.store_scatter(tbuf, [jnp.broadcast_to(48, (_L,)) + iv],
                               jnp.where(iv < 15, 0.0, inf))
            plsc.store_scatter(tbuf, [iv + 1], c0)
            plsc.store_scatter(tbuf, [iv + 17], c1)
            plsc.store_scatter(tbuf, [iv + 33], c2)
            plsc.store_scatter(tbuf, [iv + 49], c3, mask=iv < 14)
            for t in range(4):
                za = plsc.load_gather(zbuf, [jnp.broadcast_to(rb + t * _L, (_L,)) + iv])
                zb = plsc.load_gather(zbuf, [jnp.broadcast_to(rb + 1 + t * _L, (_L,)) + iv])
                plsc.store_scatter(mbuf, [jnp.broadcast_to(t * _L, (_L,)) + iv],
                                   0.5 * (za + zb))
            sv = jnp.broadcast_to(s_tot, (_L,))
            thr = jnp.broadcast_to(1e-5 * s_tot, (_L,))
            for c in range(_NI // _L):
                us = u_vecs[c] * sv
                cnt = jnp.zeros((_L,), _I32)
                for step in (32, 16, 8, 4, 2, 1):
                    t2 = cnt + step
                    val = plsc.load_gather(tbuf, [t2 - 1])
                    cnt = jnp.where(val <= us, t2, cnt)
                below = cnt - 1
                above = jnp.minimum(cnt, 62)
                g0 = plsc.load_gather(tbuf, [below])
                g1 = plsc.load_gather(tbuf, [above])
                b0 = plsc.load_gather(mbuf, [below])
                b1 = plsc.load_gather(mbuf, [above])
                den = g1 - g0
                num = us - g0
                tt = num / jnp.where(den < thr, sv, den)
                zs = b0 + tt * (b1 - b0)
                plsc.store_scatter(
                    zsbuf, [jnp.broadcast_to(r * _NI + c * _L, (_L,)) + iv], zs)

        lax.fori_loop(0, _CH, do_ray, None)
        pltpu.sync_copy(zsbuf.at[pl.ds(0, _CH * _NI)],
                        zs_hbm.at[pl.ds(base * _NI, _CH * _NI)])

    lax.fori_loop(0, rpw // _CH, do_chunk, None)


def _sc_resample(z_flat, w_flat, u, n_rays):
    mesh = plsc.VectorSubcoreMesh(core_axis_name="c", subcore_axis_name="s")
    f = functools.partial(
        pl.kernel,
        out_type=jax.ShapeDtypeStruct((n_rays * _NI,), _F32),
        mesh=mesh,
        compiler_params=pltpu.CompilerParams(needs_layout_passes=False),
        scratch_types=[
            pltpu.VMEM((_CH * _NS + _L,), _F32),
            pltpu.VMEM((_CH * _NS + _L,), _F32),
            pltpu.VMEM((_NI,), _F32),
            pltpu.VMEM((80,), _F32),
            pltpu.VMEM((_NS,), _F32),
            pltpu.VMEM((_CH * _NI,), _F32),
            pltpu.SemaphoreType.DMA,
        ],
    )(functools.partial(_sc_resample_body, n_rays // _NW))
    return f(z_flat, w_flat, u)


def _bitonic_stages(x, width, kbs, descending=False):
    col = lax.broadcasted_iota(_I32, (1, width), 1)
    for kb in kbs:
        kk = 1 << kb
        jj = kk >> 1
        while jj >= 1:
            low = (col & jj) == 0
            partner = jnp.where(low, jnp.roll(x, -jj, axis=1),
                                jnp.roll(x, jj, axis=1))
            keepmin = low == (((col & kk) == 0) != descending)
            x = jnp.where(keepmin, jnp.minimum(x, partner),
                          jnp.maximum(x, partner))
            jj >>= 1
    return x


def _tc_presort_body(z_ref, zsort_ref):
    zsort_ref[...] = _bitonic_stages(z_ref[...], _NS, range(1, 7),
                                     descending=True)


def _tc_presort(z_vals):
    n = z_vals.shape[0]
    return pl.pallas_call(
        _tc_presort_body,
        grid=(n // _B,),
        in_specs=[pl.BlockSpec((_B, _NS), lambda i: (i, 0))],
        out_specs=pl.BlockSpec((_B, _NS), lambda i: (i, 0)),
        out_shape=jax.ShapeDtypeStruct((n, _NS), _F32),
    )(z_vals)


def _tc_sort_pts_body(o_ref, d_ref, z_ref, zs_ref, pts_ref, zall_ref):
    B = z_ref.shape[0]
    z = z_ref[...]
    zs = zs_ref[...]

    zs = _bitonic_stages(zs, _NI, range(1, 8))
    x = jnp.concatenate(
        [zs, jnp.full((B, _NP - _NT), jnp.inf, _F32), z], axis=1)
    x = _bitonic_stages(x, _NP, [8])
    z_all = x[:, :_NT]
    zall_ref[...] = z_all

    rp = lax.broadcasted_iota(_I32, (_NT, 3 * _NT), 0)
    cp = lax.broadcasted_iota(_I32, (_NT, 3 * _NT), 1)
    P = jnp.where(rp == cp // 3, 1.0, 0.0).astype(_F32)
    z3 = lax.dot_general(z_all, P, (((1,), (0,)), ((), ())),
                         precision=lax.Precision.HIGHEST,
                         preferred_element_type=_F32)

    c3 = lax.broadcasted_iota(_I32, (1, 3 * _NT), 1) % 3

    def expand(a):
        return jnp.where(c3 == 0, a[:, 0:1],
                         jnp.where(c3 == 1, a[:, 1:2], a[:, 2:3]))

    pts_ref[...] = expand(o_ref[...]) + expand(d_ref[...]) * z3


def _tc_sort_pts(rays_o, rays_d, z_vals, zs):
    n = z_vals.shape[0]
    return pl.pallas_call(
        _tc_sort_pts_body,
        grid=(n // _B,),
        in_specs=[
            pl.BlockSpec((_B, 3), lambda i: (i, 0)),
            pl.BlockSpec((_B, 3), lambda i: (i, 0)),
            pl.BlockSpec((_B, _NS), lambda i: (i, 0)),
            pl.BlockSpec((_B, _NI), lambda i: (i, 0)),
        ],
        out_specs=[
            pl.BlockSpec((_B, 3 * _NT), lambda i: (i, 0)),
            pl.BlockSpec((_B, _NT), lambda i: (i, 0)),
        ],
        out_shape=[
            jax.ShapeDtypeStruct((n, 3 * _NT), _F32),
            jax.ShapeDtypeStruct((n, _NT), _F32),
        ],
    )(rays_o, rays_d, z_vals, zs)


def kernel(rays_o, rays_d, z_vals, weights):
    n = z_vals.shape[0]
    h = n // 2
    u = jnp.linspace(0.0, 1.0, _NI, dtype=_F32)
    zs0 = _sc_resample(z_vals[:h].reshape(-1), weights[:h].reshape(-1), u, h)
    zs1 = _sc_resample(z_vals[h:].reshape(-1), weights[h:].reshape(-1), u, h)
    z_sorted = _tc_presort(z_vals)
    p0, za0 = _tc_sort_pts(rays_o[:h], rays_d[:h], z_sorted[:h],
                           zs0.reshape(h, _NI))
    p1, za1 = _tc_sort_pts(rays_o[h:], rays_d[h:], z_sorted[h:],
                           zs1.reshape(h, _NI))
    pts_flat = jnp.concatenate([p0, p1], axis=0)
    z_all = jnp.concatenate([za0, za1], axis=0)
    return pts_flat.reshape(n, _NT, 3), z_all


# speedup vs baseline: 1.2696x; 1.2696x over previous
import functools

import jax
import jax.numpy as jnp
from jax import lax
from jax.experimental import pallas as pl
from jax.experimental.pallas import tpu as pltpu
from jax.experimental.pallas import tpu_sc as plsc

_NS = 64
_NI = 128
_NT = _NS + _NI
_NP = 256
_B = 256

_F32 = jnp.float32
_I32 = jnp.int32

_N_RAYS = 16384
_NW = 32
_RPW = _N_RAYS // _NW
_CH = 256
_L = 16


def _sc_resample_body(rpw, z_hbm, w_hbm, u_hbm, zs_hbm,
                      zbuf, wbuf, ubuf, tbuf, mbuf, zsbuf, sem):
    nc = 2
    wid = lax.axis_index("s") * nc + lax.axis_index("c")
    iv = lax.iota(_I32, _L)
    inf = jnp.full((_L,), jnp.inf, _F32)

    pltpu.sync_copy(u_hbm, ubuf)
    u_vecs = [ubuf[pl.ds(c * _L, _L)] for c in range(_NI // _L)]

    def do_chunk(chunk, _):
        base = wid * rpw + chunk * _CH
        pltpu.sync_copy(z_hbm.at[pl.ds(base * _NS, _CH * _NS)],
                        zbuf.at[pl.ds(0, _CH * _NS)])
        pltpu.sync_copy(w_hbm.at[pl.ds(base * _NS, _CH * _NS)],
                        wbuf.at[pl.ds(0, _CH * _NS)])

        def do_ray(r, _):
            rb = r * _NS
            wv = []
            for t in range(4):
                wt = plsc.load_gather(wbuf, [jnp.broadcast_to(rb + 1 + t * _L, (_L,)) + iv])
                wt = wt + 1e-5
                if t == 3:
                    wt = jnp.where(iv < 14, wt, 0.0)
                wv.append(wt)
            s0 = jnp.sum(wv[0])
            s1 = jnp.sum(wv[1])
            s2 = jnp.sum(wv[2])
            s3 = jnp.sum(wv[3])
            c0 = plsc.cumsum(wv[0])
            c1 = plsc.cumsum(wv[1]) + s0
            c2 = plsc.cumsum(wv[2]) + (s0 + s1)
            c3 = plsc.cumsum(wv[3]) + (s0 + s1 + s2)
            s_tot = s0 + s1 + s2 + s3
            plsc.store_scatter(tbuf, [iv], jnp.where(iv == 0, 0.0, inf))
            plsc.store_scatter(tbuf, [jnp.broadcast_to(48, (_L,)) + iv],
                               jnp.where(iv < 15, 0.0, inf))
            plsc.store_scatter(tbuf, [iv + 1], c0)
            plsc.store_scatter(tbuf, [iv + 17], c1)
            plsc.store_scatter(tbuf, [iv + 33], c2)
            plsc.store_scatter(tbuf, [iv + 49], c3, mask=iv < 14)
            for t in range(4):
                za = plsc.load_gather(zbuf, [jnp.broadcast_to(rb + t * _L, (_L,)) + iv])
                zb = plsc.load_gather(zbuf, [jnp.broadcast_to(rb + 1 + t * _L, (_L,)) + iv])
                plsc.store_scatter(mbuf, [jnp.broadcast_to(t * _L, (_L,)) + iv],
                                   0.5 * (za + zb))
            sv = jnp.broadcast_to(s_tot, (_L,))
            thr = jnp.broadcast_to(1e-5 * s_tot, (_L,))
            for c in range(_NI // _L):
                us = u_vecs[c] * sv
                cnt = jnp.zeros((_L,), _I32)
                for step in (32, 16, 8, 4, 2, 1):
                    t2 = cnt + step
                    val = plsc.load_gather(tbuf, [t2 - 1])
                    cnt = jnp.where(val <= us, t2, cnt)
                below = cnt - 1
                above = jnp.minimum(cnt, 62)
                g0 = plsc.load_gather(tbuf, [below])
                g1 = plsc.load_gather(tbuf, [above])
                b0 = plsc.load_gather(mbuf, [below])
                b1 = plsc.load_gather(mbuf, [above])
                den = g1 - g0
                num = us - g0
                tt = num / jnp.where(den < thr, sv, den)
                zs = b0 + tt * (b1 - b0)
                plsc.store_scatter(
                    zsbuf, [jnp.broadcast_to(r * _NI + c * _L, (_L,)) + iv], zs)

        lax.fori_loop(0, _CH, do_ray, None)
        pltpu.sync_copy(zsbuf.at[pl.ds(0, _CH * _NI)],
                        zs_hbm.at[pl.ds(base * _NI, _CH * _NI)])

    lax.fori_loop(0, rpw // _CH, do_chunk, None)


def _sc_resample(z_flat, w_flat, u, n_rays):
    mesh = plsc.VectorSubcoreMesh(core_axis_name="c", subcore_axis_name="s")
    f = functools.partial(
        pl.kernel,
        out_type=jax.ShapeDtypeStruct((n_rays * _NI,), _F32),
        mesh=mesh,
        compiler_params=pltpu.CompilerParams(needs_layout_passes=False),
        scratch_types=[
            pltpu.VMEM((_CH * _NS + _L,), _F32),
            pltpu.VMEM((_CH * _NS + _L,), _F32),
            pltpu.VMEM((_NI,), _F32),
            pltpu.VMEM((80,), _F32),
            pltpu.VMEM((_NS,), _F32),
            pltpu.VMEM((_CH * _NI,), _F32),
            pltpu.SemaphoreType.DMA,
        ],
    )(functools.partial(_sc_resample_body, n_rays // _NW))
    return f(z_flat, w_flat, u)


def _bitonic_stages(x, width, kbs, descending=False, block=None):
    col = lax.broadcasted_iota(_I32, (1, width), 1)
    if block is not None:
        col = col & (block - 1)
    for kb in kbs:
        kk = 1 << kb
        jj = kk >> 1
        while jj >= 1:
            low = (col & jj) == 0
            partner = jnp.where(low, jnp.roll(x, -jj, axis=1),
                                jnp.roll(x, jj, axis=1))
            keepmin = low == (((col & kk) == 0) != descending)
            x = jnp.where(keepmin, jnp.minimum(x, partner),
                          jnp.maximum(x, partner))
            jj >>= 1
    return x


def _tc_presort_body(z_ref, zsort_ref):
    zsort_ref[...] = _bitonic_stages(z_ref[...], 4 * _NS, range(1, 7),
                                     descending=True, block=_NS)


def _tc_presort(z_vals):
    n = z_vals.shape[0]
    z4 = z_vals.reshape(n // 4, 4 * _NS)
    out = pl.pallas_call(
        _tc_presort_body,
        grid=(n // 4 // _B,),
        in_specs=[pl.BlockSpec((_B, 4 * _NS), lambda i: (i, 0))],
        out_specs=pl.BlockSpec((_B, 4 * _NS), lambda i: (i, 0)),
        out_shape=jax.ShapeDtypeStruct((n // 4, 4 * _NS), _F32),
    )(z4)
    return out.reshape(n, _NS)


def _tc_sort_pts_body(o_ref, d_ref, z_ref, zs_ref, pts_ref, zall_ref):
    B = z_ref.shape[0]
    z = z_ref[...]
    zs = zs_ref[...]

    zs = _bitonic_stages(zs, _NI, range(1, 8))
    x = jnp.concatenate(
        [zs, jnp.full((B, _NP - _NT), jnp.inf, _F32), z], axis=1)
    x = _bitonic_stages(x, _NP, [8])
    z_all = x[:, :_NT]
    zall_ref[...] = z_all

    rp = lax.broadcasted_iota(_I32, (_NT, 3 * _NT), 0)
    cp = lax.broadcasted_iota(_I32, (_NT, 3 * _NT), 1)
    P = jnp.where(rp == cp // 3, 1.0, 0.0).astype(_F32)
    z3 = lax.dot_general(z_all, P, (((1,), (0,)), ((), ())),
                         precision=lax.Precision.HIGHEST,
                         preferred_element_type=_F32)

    c3 = lax.broadcasted_iota(_I32, (1, 3 * _NT), 1) % 3

    def expand(a):
        return jnp.where(c3 == 0, a[:, 0:1],
                         jnp.where(c3 == 1, a[:, 1:2], a[:, 2:3]))

    pts_ref[...] = expand(o_ref[...]) + expand(d_ref[...]) * z3


def _tc_sort_pts_carry_body(o_ref, d_ref, z_ref, zs_ref, ptsc_ref, zallc_ref,
                            pts_ref, zall_ref):
    _tc_sort_pts_body(o_ref, d_ref, z_ref, zs_ref, pts_ref, zall_ref)


def _tc_sort_pts(rays_o, rays_d, z_vals, zs, n_total, block_off, carry=None):
    h = z_vals.shape[0]
    in_specs = [
        pl.BlockSpec((_B, 3), lambda i: (i, 0)),
        pl.BlockSpec((_B, 3), lambda i: (i, 0)),
        pl.BlockSpec((_B, _NS), lambda i: (i, 0)),
        pl.BlockSpec((_B, _NI), lambda i: (i, 0)),
    ]
    args = [rays_o, rays_d, z_vals, zs]
    body = _tc_sort_pts_body
    io_aliases = {}
    if carry is not None:
        in_specs += [pl.BlockSpec(memory_space=pltpu.MemorySpace.HBM)] * 2
        args += list(carry)
        io_aliases = {4: 0, 5: 1}
        body = _tc_sort_pts_carry_body
    return pl.pallas_call(
        body,
        grid=(h // _B,),
        in_specs=in_specs,
        out_specs=[
            pl.BlockSpec((_B, 3 * _NT), lambda i, o=block_off: (i + o, 0)),
            pl.BlockSpec((_B, _NT), lambda i, o=block_off: (i + o, 0)),
        ],
        out_shape=[
            jax.ShapeDtypeStruct((n_total, 3 * _NT), _F32),
            jax.ShapeDtypeStruct((n_total, _NT), _F32),
        ],
        input_output_aliases=io_aliases,
    )(*args)


def kernel(rays_o, rays_d, z_vals, weights):
    n = z_vals.shape[0]
    h = n // 2
    u = jnp.linspace(0.0, 1.0, _NI, dtype=_F32)
    zs0 = _sc_resample(z_vals[:h].reshape(-1), weights[:h].reshape(-1), u, h)
    zs1 = _sc_resample(z_vals[h:].reshape(-1), weights[h:].reshape(-1), u, h)
    z_sorted = _tc_presort(z_vals)
    p0, za0 = _tc_sort_pts(rays_o[:h], rays_d[:h], z_sorted[:h],
                           zs0.reshape(h, _NI), n, 0)
    pts_flat, z_all = _tc_sort_pts(rays_o[h:], rays_d[h:], z_sorted[h:],
                                   zs1.reshape(h, _NI), n, h // _B,
                                   carry=(p0, za0))
    return pts_flat.reshape(n, _NT, 3), z_all


# speedup vs baseline: 1.3009x; 1.0246x over previous
import functools

import jax
import jax.numpy as jnp
from jax import lax
from jax.experimental import pallas as pl
from jax.experimental.pallas import tpu as pltpu
from jax.experimental.pallas import tpu_sc as plsc

_NS = 64
_NI = 128
_NT = _NS + _NI
_NP = 256
_B = 256

_F32 = jnp.float32
_I32 = jnp.int32

_N_RAYS = 16384
_NW = 32
_RPW = _N_RAYS // _NW
_CH = 256
_L = 16


def _sc_resample_body(rpw, z_hbm, w_hbm, u_hbm, zs_hbm,
                      zbuf, wbuf, ubuf, tbuf, mbuf, zsbuf, sem):
    nc = 2
    wid = lax.axis_index("s") * nc + lax.axis_index("c")
    iv = lax.iota(_I32, _L)
    inf = jnp.full((_L,), jnp.inf, _F32)

    pltpu.sync_copy(u_hbm, ubuf)
    u_vecs = [ubuf[pl.ds(c * _L, _L)] for c in range(_NI // _L)]

    plsc.store_scatter(tbuf, [iv], jnp.where(iv == 0, 0.0, inf))
    plsc.store_scatter(tbuf, [jnp.broadcast_to(48, (_L,)) + iv],
                       jnp.where(iv < 15, 0.0, inf))
    plsc.store_scatter(tbuf, [jnp.broadcast_to(64, (_L,)) + iv], inf)

    def do_chunk(chunk, _):
        base = wid * rpw + chunk * _CH
        pltpu.sync_copy(z_hbm.at[pl.ds(base * _NS, _CH * _NS)],
                        zbuf.at[pl.ds(0, _CH * _NS)])
        pltpu.sync_copy(w_hbm.at[pl.ds(base * _NS, _CH * _NS)],
                        wbuf.at[pl.ds(0, _CH * _NS)])

        def do_ray(r, _):
            rb = r * _NS
            wv = []
            for t in range(4):
                wt = plsc.load_gather(wbuf, [jnp.broadcast_to(rb + 1 + t * _L, (_L,)) + iv])
                wt = wt + 1e-5
                if t == 3:
                    wt = jnp.where(iv < 14, wt, 0.0)
                wv.append(wt)
            s0 = jnp.sum(wv[0])
            s1 = jnp.sum(wv[1])
            s2 = jnp.sum(wv[2])
            s3 = jnp.sum(wv[3])
            c0 = plsc.cumsum(wv[0])
            c1 = plsc.cumsum(wv[1]) + s0
            c2 = plsc.cumsum(wv[2]) + (s0 + s1)
            c3 = plsc.cumsum(wv[3]) + (s0 + s1 + s2)
            s_tot = s0 + s1 + s2 + s3
            plsc.store_scatter(tbuf, [iv + 1], c0)
            plsc.store_scatter(tbuf, [iv + 17], c1)
            plsc.store_scatter(tbuf, [iv + 33], c2)
            plsc.store_scatter(tbuf, [iv + 49], c3, mask=iv < 14)
            for t in range(4):
                za = plsc.load_gather(zbuf, [jnp.broadcast_to(rb + t * _L, (_L,)) + iv])
                zb = plsc.load_gather(zbuf, [jnp.broadcast_to(rb + 1 + t * _L, (_L,)) + iv])
                plsc.store_scatter(mbuf, [jnp.broadcast_to(t * _L, (_L,)) + iv],
                                   0.5 * (za + zb))
            sv = jnp.broadcast_to(s_tot, (_L,))
            thr = jnp.broadcast_to(1e-5 * s_tot, (_L,))
            for c in range(_NI // _L):
                us = u_vecs[c] * sv
                cnt = jnp.zeros((_L,), _I32)
                g0 = jnp.zeros((_L,), _F32)
                g1 = sv
                for step in (32, 16, 8, 4, 2, 1):
                    t2 = cnt + step
                    val = plsc.load_gather(tbuf, [t2 - 1])
                    acc = val <= us
                    cnt = jnp.where(acc, t2, cnt)
                    g0 = jnp.where(acc, val, g0)
                    g1 = jnp.where(acc, g1, val)
                below = cnt - 1
                above = jnp.minimum(cnt, 62)
                b0 = plsc.load_gather(mbuf, [below])
                b1 = plsc.load_gather(mbuf, [above])
                den = g1 - g0
                num = us - g0
                tt = num / jnp.where(den < thr, sv, den)
                zs = b0 + tt * (b1 - b0)
                plsc.store_scatter(
                    zsbuf, [jnp.broadcast_to(r * _NI + c * _L, (_L,)) + iv], zs)

        lax.fori_loop(0, _CH, do_ray, None)
        pltpu.sync_copy(zsbuf.at[pl.ds(0, _CH * _NI)],
                        zs_hbm.at[pl.ds(base * _NI, _CH * _NI)])

    lax.fori_loop(0, rpw // _CH, do_chunk, None)


def _sc_resample(z_flat, w_flat, u, n_rays):
    mesh = plsc.VectorSubcoreMesh(core_axis_name="c", subcore_axis_name="s")
    f = functools.partial(
        pl.kernel,
        out_type=jax.ShapeDtypeStruct((n_rays * _NI,), _F32),
        mesh=mesh,
        compiler_params=pltpu.CompilerParams(needs_layout_passes=False),
        scratch_types=[
            pltpu.VMEM((_CH * _NS + _L,), _F32),
            pltpu.VMEM((_CH * _NS + _L,), _F32),
            pltpu.VMEM((_NI,), _F32),
            pltpu.VMEM((80,), _F32),
            pltpu.VMEM((_NS,), _F32),
            pltpu.VMEM((_CH * _NI,), _F32),
            pltpu.SemaphoreType.DMA,
        ],
    )(functools.partial(_sc_resample_body, n_rays // _NW))
    return f(z_flat, w_flat, u)


def _bitonic_stages(x, width, kbs, descending=False, block=None):
    col = lax.broadcasted_iota(_I32, (1, width), 1)
    if block is not None:
        col = col & (block - 1)
    for kb in kbs:
        kk = 1 << kb
        jj = kk >> 1
        while jj >= 1:
            low = (col & jj) == 0
            partner = jnp.where(low, jnp.roll(x, -jj, axis=1),
                                jnp.roll(x, jj, axis=1))
            keepmin = low == (((col & kk) == 0) != descending)
            x = jnp.where(keepmin, jnp.minimum(x, partner),
                          jnp.maximum(x, partner))
            jj >>= 1
    return x


def _tc_presort_body(z_ref, zsort_ref):
    zsort_ref[...] = _bitonic_stages(z_ref[...], 4 * _NS, range(1, 7),
                                     descending=True, block=_NS)


def _tc_presort(z_vals):
    n = z_vals.shape[0]
    z4 = z_vals.reshape(n // 4, 4 * _NS)
    out = pl.pallas_call(
        _tc_presort_body,
        grid=(n // 4 // _B,),
        in_specs=[pl.BlockSpec((_B, 4 * _NS), lambda i: (i, 0))],
        out_specs=pl.BlockSpec((_B, 4 * _NS), lambda i: (i, 0)),
        out_shape=jax.ShapeDtypeStruct((n // 4, 4 * _NS), _F32),
    )(z4)
    return out.reshape(n, _NS)


def _tc_sort_pts_body(o_ref, d_ref, z_ref, zs_ref, pts_ref, zall_ref):
    B = z_ref.shape[0]
    z = z_ref[...]
    zs = zs_ref[...]

    zs = _bitonic_stages(zs, _NI, range(1, 8))
    x = jnp.concatenate(
        [zs, jnp.full((B, _NP - _NT), jnp.inf, _F32), z], axis=1)
    x = _bitonic_stages(x, _NP, [8])
    z_all = x[:, :_NT]
    zall_ref[...] = z_all

    rp = lax.broadcasted_iota(_I32, (_NT, 3 * _NT), 0)
    cp = lax.broadcasted_iota(_I32, (_NT, 3 * _NT), 1)
    P = jnp.where(rp == cp // 3, 1.0, 0.0).astype(_F32)
    z3 = lax.dot_general(z_all, P, (((1,), (0,)), ((), ())),
                         precision=lax.Precision.HIGHEST,
                         preferred_element_type=_F32)

    c3 = lax.broadcasted_iota(_I32, (1, 3 * _NT), 1) % 3

    def expand(a):
        return jnp.where(c3 == 0, a[:, 0:1],
                         jnp.where(c3 == 1, a[:, 1:2], a[:, 2:3]))

    pts_ref[...] = expand(o_ref[...]) + expand(d_ref[...]) * z3


def _tc_sort_pts_carry_body(o_ref, d_ref, z_ref, zs_ref, ptsc_ref, zallc_ref,
                            pts_ref, zall_ref):
    _tc_sort_pts_body(o_ref, d_ref, z_ref, zs_ref, pts_ref, zall_ref)


def _tc_sort_pts(rays_o, rays_d, z_vals, zs, n_total, block_off, carry=None):
    h = z_vals.shape[0]
    in_specs = [
        pl.BlockSpec((_B, 3), lambda i: (i, 0)),
        pl.BlockSpec((_B, 3), lambda i: (i, 0)),
        pl.BlockSpec((_B, _NS), lambda i: (i, 0)),
        pl.BlockSpec((_B, _NI), lambda i: (i, 0)),
    ]
    args = [rays_o, rays_d, z_vals, zs]
    body = _tc_sort_pts_body
    io_aliases = {}
    if carry is not None:
        in_specs += [pl.BlockSpec(memory_space=pltpu.MemorySpace.HBM)] * 2
        args += list(carry)
        io_aliases = {4: 0, 5: 1}
        body = _tc_sort_pts_carry_body
    return pl.pallas_call(
        body,
        grid=(h // _B,),
        in_specs=in_specs,
        out_specs=[
            pl.BlockSpec((_B, 3 * _NT), lambda i, o=block_off: (i + o, 0)),
            pl.BlockSpec((_B, _NT), lambda i, o=block_off: (i + o, 0)),
        ],
        out_shape=[
            jax.ShapeDtypeStruct((n_total, 3 * _NT), _F32),
            jax.ShapeDtypeStruct((n_total, _NT), _F32),
        ],
        input_output_aliases=io_aliases,
    )(*args)


def kernel(rays_o, rays_d, z_vals, weights):
    n = z_vals.shape[0]
    h = n // 2
    u = jnp.linspace(0.0, 1.0, _NI, dtype=_F32)
    zs0 = _sc_resample(z_vals[:h].reshape(-1), weights[:h].reshape(-1), u, h)
    zs1 = _sc_resample(z_vals[h:].reshape(-1), weights[h:].reshape(-1), u, h)
    z_sorted = _tc_presort(z_vals)
    p0, za0 = _tc_sort_pts(rays_o[:h], rays_d[:h], z_sorted[:h],
                           zs0.reshape(h, _NI), n, 0)
    pts_flat, z_all = _tc_sort_pts(rays_o[h:], rays_d[h:], z_sorted[h:],
                                   zs1.reshape(h, _NI), n, h // _B,
                                   carry=(p0, za0))
    return pts_flat.reshape(n, _NT, 3), z_all


# speedup vs baseline: 1.3010x; 1.0001x over previous
import functools

import jax
import jax.numpy as jnp
from jax import lax
from jax.experimental import pallas as pl
from jax.experimental.pallas import tpu as pltpu
from jax.experimental.pallas import tpu_sc as plsc

_NS = 64
_NI = 128
_NT = _NS + _NI
_NP = 256
_B = 256

_F32 = jnp.float32
_I32 = jnp.int32

_N_RAYS = 16384
_NW = 32
_RPW = _N_RAYS // _NW
_CH = 256
_L = 16


def _sc_resample_body(rpw, z_hbm, w_hbm, u_hbm, zs_hbm,
                      zbuf, wbuf, ubuf, tbuf, mbuf, zsbuf, sem):
    nc = 2
    wid = lax.axis_index("s") * nc + lax.axis_index("c")
    iv = lax.iota(_I32, _L)
    inf = jnp.full((_L,), jnp.inf, _F32)

    pltpu.sync_copy(u_hbm, ubuf)
    u_vecs = [ubuf[pl.ds(c * _L, _L)] for c in range(_NI // _L)]

    plsc.store_scatter(tbuf, [iv], jnp.where(iv == 0, 0.0, inf))
    plsc.store_scatter(tbuf, [jnp.broadcast_to(48, (_L,)) + iv],
                       jnp.where(iv < 15, 0.0, inf))
    plsc.store_scatter(tbuf, [jnp.broadcast_to(64, (_L,)) + iv], inf)

    def do_chunk(chunk, _):
        base = wid * rpw + chunk * _CH
        pltpu.sync_copy(z_hbm.at[pl.ds(base * _NS, _CH * _NS)],
                        zbuf.at[pl.ds(0, _CH * _NS)])
        pltpu.sync_copy(w_hbm.at[pl.ds(base * _NS, _CH * _NS)],
                        wbuf.at[pl.ds(0, _CH * _NS)])

        def do_ray(r, _):
            rb = r * _NS
            wv = []
            for t in range(4):
                wt = wbuf[pl.ds(rb + 1 + t * _L, _L)]
                wt = wt + 1e-5
                if t == 3:
                    wt = jnp.where(iv < 14, wt, 0.0)
                wv.append(wt)
            s0 = jnp.sum(wv[0])
            s1 = jnp.sum(wv[1])
            s2 = jnp.sum(wv[2])
            s3 = jnp.sum(wv[3])
            c0 = plsc.cumsum(wv[0])
            c1 = plsc.cumsum(wv[1]) + s0
            c2 = plsc.cumsum(wv[2]) + (s0 + s1)
            c3 = plsc.cumsum(wv[3]) + (s0 + s1 + s2)
            s_tot = s0 + s1 + s2 + s3
            plsc.store_scatter(tbuf, [iv + 1], c0)
            plsc.store_scatter(tbuf, [iv + 17], c1)
            plsc.store_scatter(tbuf, [iv + 33], c2)
            plsc.store_scatter(tbuf, [iv + 49], c3, mask=iv < 14)
            for t in range(4):
                za = zbuf[pl.ds(rb + t * _L, _L)]
                zb = zbuf[pl.ds(rb + 1 + t * _L, _L)]
                mbuf[pl.ds(t * _L, _L)] = 0.5 * (za + zb)
            sv = jnp.broadcast_to(s_tot, (_L,))
            thr = jnp.broadcast_to(1e-5 * s_tot, (_L,))
            for c in range(_NI // _L):
                us = u_vecs[c] * sv
                cnt = jnp.zeros((_L,), _I32)
                g0 = jnp.zeros((_L,), _F32)
                g1 = sv
                for step in (32, 16, 8, 4, 2, 1):
                    t2 = cnt + step
                    val = plsc.load_gather(tbuf, [t2 - 1])
                    acc = val <= us
                    cnt = jnp.where(acc, t2, cnt)
                    g0 = jnp.where(acc, val, g0)
                    g1 = jnp.where(acc, g1, val)
                below = cnt - 1
                above = jnp.minimum(cnt, 62)
                b0 = plsc.load_gather(mbuf, [below])
                b1 = plsc.load_gather(mbuf, [above])
                den = g1 - g0
                num = us - g0
                tt = num / jnp.where(den < thr, sv, den)
                zs = b0 + tt * (b1 - b0)
                zsbuf[pl.ds(r * _NI + c * _L, _L)] = zs

        lax.fori_loop(0, _CH, do_ray, None)
        pltpu.sync_copy(zsbuf.at[pl.ds(0, _CH * _NI)],
                        zs_hbm.at[pl.ds(base * _NI, _CH * _NI)])

    lax.fori_loop(0, rpw // _CH, do_chunk, None)


def _sc_resample(z_flat, w_flat, u, n_rays):
    mesh = plsc.VectorSubcoreMesh(core_axis_name="c", subcore_axis_name="s")
    f = functools.partial(
        pl.kernel,
        out_type=jax.ShapeDtypeStruct((n_rays * _NI,), _F32),
        mesh=mesh,
        compiler_params=pltpu.CompilerParams(needs_layout_passes=False),
        scratch_types=[
            pltpu.VMEM((_CH * _NS + _L,), _F32),
            pltpu.VMEM((_CH * _NS + _L,), _F32),
            pltpu.VMEM((_NI,), _F32),
            pltpu.VMEM((80,), _F32),
            pltpu.VMEM((_NS,), _F32),
            pltpu.VMEM((_CH * _NI,), _F32),
            pltpu.SemaphoreType.DMA,
        ],
    )(functools.partial(_sc_resample_body, n_rays // _NW))
    return f(z_flat, w_flat, u)


def _bitonic_stages(x, width, kbs, descending=False, block=None):
    col = lax.broadcasted_iota(_I32, (1, width), 1)
    if block is not None:
        col = col & (block - 1)
    for kb in kbs:
        kk = 1 << kb
        jj = kk >> 1
        while jj >= 1:
            low = (col & jj) == 0
            partner = jnp.where(low, jnp.roll(x, -jj, axis=1),
                                jnp.roll(x, jj, axis=1))
            keepmin = low == (((col & kk) == 0) != descending)
            x = jnp.where(keepmin, jnp.minimum(x, partner),
                          jnp.maximum(x, partner))
            jj >>= 1
    return x


def _tc_presort_body(z_ref, zsort_ref):
    zsort_ref[...] = _bitonic_stages(z_ref[...], 4 * _NS, range(1, 7),
                                     descending=True, block=_NS)


def _tc_presort(z_vals):
    n = z_vals.shape[0]
    z4 = z_vals.reshape(n // 4, 4 * _NS)
    out = pl.pallas_call(
        _tc_presort_body,
        grid=(n // 4 // _B,),
        in_specs=[pl.BlockSpec((_B, 4 * _NS), lambda i: (i, 0))],
        out_specs=pl.BlockSpec((_B, 4 * _NS), lambda i: (i, 0)),
        out_shape=jax.ShapeDtypeStruct((n // 4, 4 * _NS), _F32),
    )(z4)
    return out.reshape(n, _NS)


def _tc_sort_pts_body(o_ref, d_ref, z_ref, zs_ref, pts_ref, zall_ref):
    B = z_ref.shape[0]
    z = z_ref[...]
    zs = zs_ref[...]

    zs = _bitonic_stages(zs, _NI, range(1, 8))
    x = jnp.concatenate(
        [zs, jnp.full((B, _NP - _NT), jnp.inf, _F32), z], axis=1)
    x = _bitonic_stages(x, _NP, [8])
    z_all = x[:, :_NT]
    zall_ref[...] = z_all

    rp = lax.broadcasted_iota(_I32, (_NT, 3 * _NT), 0)
    cp = lax.broadcasted_iota(_I32, (_NT, 3 * _NT), 1)
    P = jnp.where(rp == cp // 3, 1.0, 0.0).astype(_F32)
    z3 = lax.dot_general(z_all, P, (((1,), (0,)), ((), ())),
                         precision=lax.Precision.HIGHEST,
                         preferred_element_type=_F32)

    c3 = lax.broadcasted_iota(_I32, (1, 3 * _NT), 1) % 3

    def expand(a):
        return jnp.where(c3 == 0, a[:, 0:1],
                         jnp.where(c3 == 1, a[:, 1:2], a[:, 2:3]))

    pts_ref[...] = expand(o_ref[...]) + expand(d_ref[...]) * z3


def _tc_sort_pts_carry_body(o_ref, d_ref, z_ref, zs_ref, ptsc_ref, zallc_ref,
                            pts_ref, zall_ref):
    _tc_sort_pts_body(o_ref, d_ref, z_ref, zs_ref, pts_ref, zall_ref)


def _tc_sort_pts(rays_o, rays_d, z_vals, zs, n_total, block_off, carry=None):
    h = z_vals.shape[0]
    in_specs = [
        pl.BlockSpec((_B, 3), lambda i: (i, 0)),
        pl.BlockSpec((_B, 3), lambda i: (i, 0)),
        pl.BlockSpec((_B, _NS), lambda i: (i, 0)),
        pl.BlockSpec((_B, _NI), lambda i: (i, 0)),
    ]
    args = [rays_o, rays_d, z_vals, zs]
    body = _tc_sort_pts_body
    io_aliases = {}
    if carry is not None:
        in_specs += [pl.BlockSpec(memory_space=pltpu.MemorySpace.HBM)] * 2
        args += list(carry)
        io_aliases = {4: 0, 5: 1}
        body = _tc_sort_pts_carry_body
    return pl.pallas_call(
        body,
        grid=(h // _B,),
        in_specs=in_specs,
        out_specs=[
            pl.BlockSpec((_B, 3 * _NT), lambda i, o=block_off: (i + o, 0)),
            pl.BlockSpec((_B, _NT), lambda i, o=block_off: (i + o, 0)),
        ],
        out_shape=[
            jax.ShapeDtypeStruct((n_total, 3 * _NT), _F32),
            jax.ShapeDtypeStruct((n_total, _NT), _F32),
        ],
        input_output_aliases=io_aliases,
    )(*args)


def kernel(rays_o, rays_d, z_vals, weights):
    n = z_vals.shape[0]
    h = n // 2
    u = jnp.linspace(0.0, 1.0, _NI, dtype=_F32)
    zs0 = _sc_resample(z_vals[:h].reshape(-1), weights[:h].reshape(-1), u, h)
    zs1 = _sc_resample(z_vals[h:].reshape(-1), weights[h:].reshape(-1), u, h)
    z_sorted = _tc_presort(z_vals)
    p0, za0 = _tc_sort_pts(rays_o[:h], rays_d[:h], z_sorted[:h],
                           zs0.reshape(h, _NI), n, 0)
    pts_flat, z_all = _tc_sort_pts(rays_o[h:], rays_d[h:], z_sorted[h:],
                                   zs1.reshape(h, _NI), n, h // _B,
                                   carry=(p0, za0))
    return pts_flat.reshape(n, _NT, 3), z_all
